# baseline (speedup 1.0000x reference)
import functools

import jax
import jax.numpy as jnp
from jax import lax
from jax.experimental import pallas as pl
from jax.experimental.pallas import tpu as pltpu
from jax.experimental.pallas import tpu_sc as plsc

VOCAB = 1000
HIDDEN = 256

NC = 2
NS = 16
NW = NC * NS

C = 128
NCH = 50
D = HIDDEN


def _proj_kernel(e_ref, w_ref, b_ref, out_ref):
    out_ref[...] = lax.dot_general(
        e_ref[...], w_ref[...],
        dimension_numbers=(((1,), (1,)), ((), ())),
        preferred_element_type=jnp.float32,
    ) + b_ref[...]


def _project_table(E, W, b2d):
    return pl.pallas_call(
        _proj_kernel,
        out_shape=jax.ShapeDtypeStruct((VOCAB, HIDDEN), jnp.float32),
    )(E, W, b2d)


def _gather_body(table_hbm, idx_hbm, out_hbm, idx_v,
                 rows0, rows1, gsem0, gsem1, ssem0, ssem1):
    wid = lax.axis_index("s") * NC + lax.axis_index("c")
    rows = (rows0, rows1)
    gsem = (gsem0, gsem1)
    ssem = (ssem0, ssem1)
    row0 = wid * (NCH * C)
    pltpu.sync_copy(idx_hbm.at[wid], idx_v)

    g0 = pltpu.async_copy(table_hbm.at[idx_v.at[0]], rows0, gsem0)
    g1 = pltpu.async_copy(table_hbm.at[idx_v.at[1]], rows1, gsem1)
    del g0, g1

    @pl.loop(0, NCH, step=2)
    def _(ch):
        for b in range(2):
            cur = ch + b
            pltpu.make_async_copy(table_hbm.at[idx_v.at[cur]],
                                  rows[b], gsem[b]).wait()
            pltpu.async_copy(rows[b], out_hbm.at[pl.ds(row0 + cur * C, C)], ssem[b])
            bp = b ^ 1
            prev = cur - 1

            @pl.when(prev >= 0)
            def _():
                pltpu.make_async_copy(rows[bp], out_hbm.at[pl.ds(row0 + prev * C, C)],
                                      ssem[bp]).wait()

            @pl.when(jnp.logical_and(cur >= 1, cur + 1 < NCH))
            def _():
                pltpu.async_copy(table_hbm.at[idx_v.at[cur + 1]],
                                 rows[bp], gsem[bp])

    pltpu.make_async_copy(rows[1], out_hbm.at[pl.ds(row0 + (NCH - 1) * C, C)],
                          ssem[1]).wait()


@functools.cache
def _gather():
    return pl.kernel(
        _gather_body,
        out_type=jax.ShapeDtypeStruct((NW * NCH * C, D), jnp.float32),
        mesh=plsc.VectorSubcoreMesh(
            core_axis_name="c", subcore_axis_name="s",
            num_cores=NC, num_subcores=NS),
        scratch_types=[
            pltpu.VMEM((NCH, C), jnp.int32),
            pltpu.VMEM((C, D), jnp.float32),
            pltpu.VMEM((C, D), jnp.float32),
            pltpu.SemaphoreType.DMA,
            pltpu.SemaphoreType.DMA,
            pltpu.SemaphoreType.DMA,
            pltpu.SemaphoreType.DMA,
        ],
    )


def kernel(input_ids, embed_table, W, b):
    P = _project_table(embed_table, W, b.reshape(1, HIDDEN))
    idx = input_ids.reshape(NW, NCH, C).astype(jnp.int32)
    out = _gather()(P, idx)
    return out.reshape(input_ids.shape[0], input_ids.shape[1], HIDDEN)


# speedup vs baseline: 1.0026x; 1.0026x over previous
import functools

import jax
import jax.numpy as jnp
from jax import lax
from jax.experimental import pallas as pl
from jax.experimental.pallas import tpu as pltpu
from jax.experimental.pallas import tpu_sc as plsc

VOCAB = 1000
HIDDEN = 256

NC = 2
NS = 16
NW = NC * NS

C = 128
NCH = 50
D = HIDDEN


def _proj_kernel(e_ref, w_ref, b_ref, out_ref):
    out_ref[...] = lax.dot_general(
        e_ref[...], w_ref[...],
        dimension_numbers=(((1,), (1,)), ((), ())),
        preferred_element_type=jnp.float32,
    ) + b_ref[...]


def _project_table(E, W, b2d):
    return pl.pallas_call(
        _proj_kernel,
        out_shape=jax.ShapeDtypeStruct((VOCAB, HIDDEN), jnp.float32),
    )(E, W, b2d)


def _gather_body(table_hbm, idx_hbm, out_hbm, idx_v,
                 rows0, rows1, gsem0, gsem1, ssem0, ssem1):
    wid = lax.axis_index("s") * NC + lax.axis_index("c")
    rows = (rows0, rows1)
    gsem = (gsem0, gsem1)
    ssem = (ssem0, ssem1)
    row0 = wid * (NCH * C)
    pltpu.sync_copy(idx_hbm.at[wid], idx_v)

    g0 = pltpu.async_copy(table_hbm.at[idx_v.at[0]], rows0, gsem0)
    g1 = pltpu.async_copy(table_hbm.at[idx_v.at[1]], rows1, gsem1)
    del g0, g1

    @pl.loop(0, NCH, step=2)
    def _(ch):
        for b in range(2):
            cur = ch + b
            pltpu.make_async_copy(table_hbm.at[idx_v.at[cur]],
                                  rows[b], gsem[b]).wait()
            pltpu.async_copy(rows[b], out_hbm.at[pl.ds(row0 + cur * C, C)], ssem[b])
            bp = b ^ 1
            prev = cur - 1

            @pl.when(prev >= 0)
            def _():
                pltpu.make_async_copy(rows[bp], out_hbm.at[pl.ds(row0 + prev * C, C)],
                                      ssem[bp]).wait()

            @pl.when(jnp.logical_and(cur >= 1, cur + 1 < NCH))
            def _():
                pltpu.async_copy(table_hbm.at[idx_v.at[cur + 1]],
                                 rows[bp], gsem[bp])

    pltpu.make_async_copy(rows[1], out_hbm.at[pl.ds(row0 + (NCH - 1) * C, C)],
                          ssem[1]).wait()


@functools.cache
def _gather():
    return pl.kernel(
        _gather_body,
        out_type=jax.ShapeDtypeStruct((NW * NCH * C, D), jnp.float32),
        mesh=plsc.VectorSubcoreMesh(
            core_axis_name="c", subcore_axis_name="s",
            num_cores=NC, num_subcores=NS),
        compiler_params=pltpu.CompilerParams(use_tc_tiling_on_sc=True),
        scratch_types=[
            pltpu.VMEM((NCH, C), jnp.int32),
            pltpu.VMEM((C, D), jnp.float32),
            pltpu.VMEM((C, D), jnp.float32),
            pltpu.SemaphoreType.DMA,
            pltpu.SemaphoreType.DMA,
            pltpu.SemaphoreType.DMA,
            pltpu.SemaphoreType.DMA,
        ],
    )


def kernel(input_ids, embed_table, W, b):
    P = _project_table(embed_table, W, b.reshape(1, HIDDEN))
    idx = input_ids.reshape(NW, NCH, C).astype(jnp.int32)
    out = _gather()(P, idx)
    return out.reshape(input_ids.shape[0], input_ids.shape[1], HIDDEN)


# speedup vs baseline: 2.6983x; 2.6912x over previous
import functools

import jax
import jax.numpy as jnp
from jax import lax
from jax.experimental import pallas as pl
from jax.experimental.pallas import tpu as pltpu
from jax.experimental.pallas import tpu_sc as plsc

VOCAB = 1000
HIDDEN = 256

NC = 2
NS = 16
NW = NC * NS

C = 128
NCH = 50
D = HIDDEN


def _proj_kernel(e_ref, w_ref, b_ref, out_ref):
    out_ref[...] = lax.dot_general(
        e_ref[...], w_ref[...],
        dimension_numbers=(((1,), (1,)), ((), ())),
        preferred_element_type=jnp.float32,
    ) + b_ref[...]


def _project_table(E, W, b2d):
    return pl.pallas_call(
        _proj_kernel,
        out_shape=jax.ShapeDtypeStruct((VOCAB, HIDDEN), jnp.float32),
    )(E, W, b2d)


def _gather_body(table_hbm, idx_hbm, out_hbm, idx_v,
                 rows0, rows1, gsem0, gsem1, ssem0, ssem1):
    wid = lax.axis_index("s") * NC + lax.axis_index("c")
    rows = (rows0, rows1)
    gsem = (gsem0, gsem1)
    ssem = (ssem0, ssem1)
    row0 = wid * (NCH * C)
    pltpu.sync_copy(idx_hbm.at[wid], idx_v)

    g0 = pltpu.async_copy(table_hbm.at[idx_v.at[0]], rows0, gsem0)
    g1 = pltpu.async_copy(table_hbm.at[idx_v.at[1]], rows1, gsem1)
    del g0, g1

    @pl.loop(0, NCH, step=2)
    def _(ch):
        for b in range(2):
            cur = ch + b
            pltpu.make_async_copy(table_hbm.at[idx_v.at[cur]],
                                  rows[b], gsem[b]).wait()
            pltpu.async_copy(rows[b], out_hbm.at[pl.ds(row0 + cur * C, C)], ssem[b])
            bp = b ^ 1
            prev = cur - 1

            @pl.when(prev >= 0)
            def _():
                pltpu.make_async_copy(rows[bp], out_hbm.at[pl.ds(row0 + prev * C, C)],
                                      ssem[bp]).wait()

            @pl.when(jnp.logical_and(cur >= 1, cur + 1 < NCH))
            def _():
                pltpu.async_copy(table_hbm.at[idx_v.at[cur + 1]],
                                 rows[bp], gsem[bp])

    pltpu.make_async_copy(rows[1], out_hbm.at[pl.ds(row0 + (NCH - 1) * C, C)],
                          ssem[1]).wait()


@functools.cache
def _gather():
    return pl.kernel(
        _gather_body,
        out_type=jax.ShapeDtypeStruct((NW * NCH * C, D), jnp.float32),
        mesh=plsc.VectorSubcoreMesh(
            core_axis_name="c", subcore_axis_name="s",
            num_cores=NC, num_subcores=NS),
        compiler_params=pltpu.CompilerParams(use_tc_tiling_on_sc=True),
        scratch_types=[
            pltpu.VMEM((NCH, C), jnp.int32),
            pltpu.VMEM((C, D), jnp.float32),
            pltpu.VMEM((C, D), jnp.float32),
            pltpu.SemaphoreType.DMA,
            pltpu.SemaphoreType.DMA,
            pltpu.SemaphoreType.DMA,
            pltpu.SemaphoreType.DMA,
        ],
    )


def kernel(input_ids, embed_table, W, b):
    B, S = input_ids.shape
    P = _project_table(embed_table, W, b.reshape(1, HIDDEN))
    idx = input_ids.T.reshape(NW, NCH, C).astype(jnp.int32)
    out = _gather()(P, idx)
    return out.reshape(S, B, HIDDEN).transpose(1, 0, 2)


# speedup vs baseline: 2.8122x; 1.0422x over previous
import functools

import jax
import jax.numpy as jnp
from jax import lax
from jax.experimental import pallas as pl
from jax.experimental.pallas import tpu as pltpu
from jax.experimental.pallas import tpu_sc as plsc

VOCAB = 1000
HIDDEN = 256

NC = 2
NS = 16
NW = NC * NS

C = 128
NCH = 50
D = HIDDEN
HC = C // 2


def _proj_kernel(e_ref, w_ref, b_ref, out_ref):
    out_ref[...] = lax.dot_general(
        e_ref[...], w_ref[...],
        dimension_numbers=(((1,), (1,)), ((), ())),
        preferred_element_type=jnp.float32,
    ) + b_ref[...]


def _project_table(E, W, b2d):
    return pl.pallas_call(
        _proj_kernel,
        out_shape=jax.ShapeDtypeStruct((VOCAB, HIDDEN), jnp.float32),
    )(E, W, b2d)


def _gather_body(table_hbm, idx_hbm, out_hbm, shr, idx_v,
                 rows0, rows1, gsem0, gsem1, bsem0, bsem1, dsem0, dsem1):
    wid = lax.axis_index("s") * NC + lax.axis_index("c")
    sid = lax.axis_index("s")
    rows = (rows0, rows1)
    gsem = (gsem0, gsem1)
    bsem = (bsem0, bsem1)
    dsem = (dsem0, dsem1)
    spm = (shr.at[sid, 0], shr.at[sid, 1])
    row0 = wid * (NCH * C)
    pltpu.sync_copy(idx_hbm.at[wid], idx_v)

    g0 = pltpu.async_copy(table_hbm.at[idx_v.at[0]], rows[0], gsem0)
    g1 = pltpu.async_copy(table_hbm.at[idx_v.at[1]], rows[1], gsem1)
    del g0, g1

    @pl.loop(0, NCH, step=2)
    def _(ch):
        for b in range(2):
            cur = ch + b
            pltpu.make_async_copy(table_hbm.at[idx_v.at[cur]],
                                  rows[b], gsem[b]).wait()

            for h in range(2):
                @pl.when(cur * 2 + h >= 2)
                def _():
                    pltpu.make_async_copy(
                        spm[h],
                        out_hbm.at[pl.ds(row0 + cur * C + (h - 2) * HC, HC)],
                        dsem[h]).wait()

                half = rows[b].at[pl.ds(h * HC, HC)]
                pltpu.async_copy(half, spm[h], bsem[h])
                pltpu.make_async_copy(half, spm[h], bsem[h]).wait()
                pltpu.async_copy(
                    spm[h],
                    out_hbm.at[pl.ds(row0 + cur * C + h * HC, HC)],
                    dsem[h])

            @pl.when(cur + 2 < NCH)
            def _():
                pltpu.async_copy(table_hbm.at[idx_v.at[cur + 2]],
                                 rows[b], gsem[b])

    for h in range(2):
        pltpu.make_async_copy(
            spm[h],
            out_hbm.at[pl.ds(row0 + (NCH - 1) * C + h * HC, HC)],
            dsem[h]).wait()


@functools.cache
def _gather():
    return pl.kernel(
        _gather_body,
        out_type=jax.ShapeDtypeStruct((NW * NCH * C, D), jnp.float32),
        mesh=plsc.VectorSubcoreMesh(
            core_axis_name="c", subcore_axis_name="s",
            num_cores=NC, num_subcores=NS),
        compiler_params=pltpu.CompilerParams(use_tc_tiling_on_sc=True),
        scratch_types=[
            pltpu.VMEM_SHARED((NS, 2, HC, D), jnp.float32),
            pltpu.VMEM((NCH, C), jnp.int32),
            pltpu.VMEM((C, D), jnp.float32),
            pltpu.VMEM((C, D), jnp.float32),
            pltpu.SemaphoreType.DMA,
            pltpu.SemaphoreType.DMA,
            pltpu.SemaphoreType.DMA,
            pltpu.SemaphoreType.DMA,
            pltpu.SemaphoreType.DMA,
            pltpu.SemaphoreType.DMA,
        ],
    )


def kernel(input_ids, embed_table, W, b):
    B, S = input_ids.shape
    P = _project_table(embed_table, W, b.reshape(1, HIDDEN))
    idx = input_ids.T.reshape(NW, NCH, C).astype(jnp.int32)
    out = _gather()(P, idx)
    return out.reshape(S, B, HIDDEN).transpose(1, 0, 2)
